# baseline (speedup 1.0000x reference)
import functools

import jax
import jax.numpy as jnp
from jax import lax
from jax.experimental import pallas as pl
from jax.experimental.pallas import tpu as pltpu
from jax.experimental.pallas import tpu_sc as plsc

N = 10000
E = 160000
D = 256
HALF = 128
NC = 2
NS = 16
EC = E // NS
G = 48
SEC = 7
BPS = 30
EP = SEC * BPS * G
NP = 10240
RPT = NP // NS
TRASH = NP - 1


def _tc_pack_x(x):
    BM = 1000

    def body(x_ref, o_ref):
        o_ref[0] = x_ref[:, :HALF]
        o_ref[1] = x_ref[:, HALF:]

    out = pl.pallas_call(
        body,
        grid=(N // BM,),
        in_specs=[pl.BlockSpec((BM, D), lambda i: (i, 0))],
        out_specs=pl.BlockSpec((2, BM, HALF), lambda i: (0, i, 0)),
        out_shape=jax.ShapeDtypeStruct((NC, NP, HALF), jnp.float32),
    )(x)
    return out.reshape(NC * NP, HALF)


def _sc_aggregate(x2, ea2, xidx5, eaidx5, dst4):
    mesh = plsc.VectorSubcoreMesh(
        core_axis_name="c", subcore_axis_name="s",
        num_cores=NC, num_subcores=NS)

    @functools.partial(
        pl.kernel,
        out_type=jax.ShapeDtypeStruct((NC * NP, HALF), jnp.float32),
        mesh=mesh,
        scratch_types=[
            pltpu.VMEM_SHARED((NP, HALF), jnp.float32),
            pltpu.VMEM((BPS, G), jnp.int32),
            pltpu.VMEM((BPS, G), jnp.int32),
            pltpu.VMEM((BPS, G), jnp.int32),
            pltpu.VMEM((2, G, HALF), jnp.float32),
            pltpu.VMEM((2, G, HALF), jnp.float32),
            pltpu.VMEM((2, G, HALF), jnp.float32),
            pltpu.SemaphoreType.DMA,
            pltpu.SemaphoreType.DMA,
            pltpu.SemaphoreType.DMA,
            pltpu.SemaphoreType.DMA,
        ],
        compiler_params=pltpu.CompilerParams(use_tc_tiling_on_sc=False),
    )
    def k(x2_hbm, ea2_hbm, xidx_hbm, eaidx_hbm, dst_hbm, out_hbm,
          acc, xidx, eaidx, dsti, xrows, earows, msg,
          sem_x, sem_e, sem_s0, sem_s1):
        c = lax.axis_index("c")
        s = lax.axis_index("s")
        base = c * NP + s * RPT

        pltpu.sync_copy(x2_hbm.at[pl.ds(base, RPT)],
                        acc.at[pl.ds(s * RPT, RPT)])
        plsc.subcore_barrier()

        def drain_scatter(p, sem):
            pltpu.make_async_copy(
                x2_hbm.at[pl.ds(0, G)], msg.at[p], sem).wait()

        def section(sec, _):
            pltpu.sync_copy(xidx_hbm.at[c, s, sec], xidx)
            pltpu.sync_copy(eaidx_hbm.at[c, s, sec], eaidx)
            pltpu.sync_copy(dst_hbm.at[s, sec], dsti)
            pltpu.async_copy(x2_hbm.at[xidx.at[0]], xrows.at[0], sem_x)
            pltpu.async_copy(ea2_hbm.at[eaidx.at[0]], earows.at[0], sem_e)

            def half_step(b, p, sem_s):
                @pl.when(b + 1 < BPS)
                def _():
                    q = 1 - p
                    pltpu.async_copy(
                        x2_hbm.at[xidx.at[b + 1]], xrows.at[q], sem_x)
                    pltpu.async_copy(
                        ea2_hbm.at[eaidx.at[b + 1]], earows.at[q], sem_e)
                pltpu.make_async_copy(
                    x2_hbm.at[xidx.at[b]], xrows.at[p], sem_x).wait()
                pltpu.make_async_copy(
                    ea2_hbm.at[eaidx.at[b]], earows.at[p], sem_e).wait()

                @pl.when(b >= 2)
                def _():
                    drain_scatter(p, sem_s)

                xr, er, mg = xrows.at[p], earows.at[p], msg.at[p]

                @plsc.parallel_loop(0, G, 1, unroll=4)
                def comp(e):
                    for kq in range(HALF // 16):
                        sl = pl.ds(kq * 16, 16)
                        mg[e, sl] = jnp.maximum(xr[e, sl] + er[e, sl], 0.0)

                pltpu.async_copy(mg, acc.at[dsti.at[b]], sem_s, add=True)

            def step(i, _):
                half_step(i * 2, 0, sem_s0)
                half_step(i * 2 + 1, 1, sem_s1)
                return 0
            lax.fori_loop(0, BPS // 2, step, 0)
            drain_scatter(0, sem_s0)
            drain_scatter(1, sem_s1)
            return 0
        lax.fori_loop(0, SEC, section, 0)

        plsc.subcore_barrier()
        pltpu.sync_copy(acc.at[pl.ds(s * RPT, RPT)],
                        out_hbm.at[pl.ds(base, RPT)])

    return k(x2, ea2, xidx5, eaidx5, dst4)


def _tc_mlp(h2, W1, b1, W2, b2):
    BM = 1000

    def body(h_ref, w1_ref, b1_ref, w2_ref, b2_ref, o_ref):
        h = jnp.dot(h_ref[0], w1_ref[:HALF, :],
                    preferred_element_type=jnp.float32)
        h = h + jnp.dot(h_ref[1], w1_ref[HALF:, :],
                        preferred_element_type=jnp.float32)
        h = jnp.maximum(h + b1_ref[0], 0.0)
        o_ref[...] = jnp.dot(h, w2_ref[...],
                             preferred_element_type=jnp.float32) + b2_ref[0]

    return pl.pallas_call(
        body,
        grid=(N // BM,),
        in_specs=[
            pl.BlockSpec((2, BM, HALF), lambda i: (0, i, 0)),
            pl.BlockSpec((D, D), lambda i: (0, 0)),
            pl.BlockSpec((1, D), lambda i: (0, 0)),
            pl.BlockSpec((D, D), lambda i: (0, 0)),
            pl.BlockSpec((1, D), lambda i: (0, 0)),
        ],
        out_specs=pl.BlockSpec((BM, D), lambda i: (i, 0)),
        out_shape=jax.ShapeDtypeStruct((N, D), jnp.float32),
    )(h2, W1, b1.reshape(1, D), W2, b2.reshape(1, D))


def kernel(x, edge_index, edge_attr, W1, b1, W2, b2):
    src = edge_index[0].astype(jnp.int32)
    dst = edge_index[1].astype(jnp.int32)
    x2 = _tc_pack_x(x)
    ea2 = edge_attr.reshape(E // 8, 8, 2, HALF).transpose(0, 2, 1, 3)
    ea2 = ea2.reshape(2 * E, HALF)
    padn = EP - EC
    srcp = jnp.pad(src.reshape(NS, EC), ((0, 0), (0, padn))).reshape(-1)
    xidx5 = (srcp[None, :] + jnp.array([[0], [NP]], jnp.int32)
             ).reshape(NC, NS, SEC, BPS, G)
    e = jnp.arange(E, dtype=jnp.int32)
    ebase = 2 * (e - (e % 8)) + (e % 8)
    ebp = jnp.pad(ebase.reshape(NS, EC), ((0, 0), (0, padn))).reshape(-1)
    eaidx5 = (ebp[None, :] + jnp.array([[0], [8]], jnp.int32)
              ).reshape(NC, NS, SEC, BPS, G)
    dst4 = jnp.pad(dst.reshape(NS, EC), ((0, 0), (0, padn)),
                   constant_values=TRASH).reshape(NS, SEC, BPS, G)
    h = _sc_aggregate(x2, ea2, xidx5, eaidx5, dst4)
    return _tc_mlp(h.reshape(NC, NP, HALF), W1, b1, W2, b2)


# speedup vs baseline: 1.0374x; 1.0374x over previous
import functools

import jax
import jax.numpy as jnp
from jax import lax
from jax.experimental import pallas as pl
from jax.experimental.pallas import tpu as pltpu
from jax.experimental.pallas import tpu_sc as plsc

N = 10000
E = 160000
D = 256
HALF = 128
NC = 2
NS = 16
EC = E // NS
G = 48
SEC = 7
BPS = 30
EP = SEC * BPS * G
NP = 10240
RPT = NP // NS
TRASH = NP - 1


def _sc_aggregate(x2, ea2, xidx5, eaidx5, dst4, zrows):
    mesh = plsc.VectorSubcoreMesh(
        core_axis_name="c", subcore_axis_name="s",
        num_cores=NC, num_subcores=NS)

    @functools.partial(
        pl.kernel,
        out_type=jax.ShapeDtypeStruct((NC * NP, HALF), jnp.float32),
        mesh=mesh,
        scratch_types=[
            pltpu.VMEM_SHARED((NP, HALF), jnp.float32),
            pltpu.VMEM((BPS, G), jnp.int32),
            pltpu.VMEM((BPS, G), jnp.int32),
            pltpu.VMEM((BPS, G), jnp.int32),
            pltpu.VMEM((2, G, HALF), jnp.float32),
            pltpu.VMEM((2, G, HALF), jnp.float32),
            pltpu.VMEM((2, G, HALF), jnp.float32),
            pltpu.SemaphoreType.DMA,
            pltpu.SemaphoreType.DMA,
            pltpu.SemaphoreType.DMA,
            pltpu.SemaphoreType.DMA,
        ],
        compiler_params=pltpu.CompilerParams(use_tc_tiling_on_sc=False),
    )
    def k(x2_hbm, ea2_hbm, xidx_hbm, eaidx_hbm, dst_hbm, z_hbm, out_hbm,
          acc, xidx, eaidx, dsti, xrows, earows, msg,
          sem_x, sem_e, sem_s0, sem_s1):
        c = lax.axis_index("c")
        s = lax.axis_index("s")
        base = c * NP + s * RPT

        pltpu.sync_copy(z_hbm, acc.at[pl.ds(s * RPT, RPT)])
        plsc.subcore_barrier()

        def drain_scatter(p, sem):
            pltpu.make_async_copy(
                x2_hbm.at[pl.ds(0, G)], msg.at[p], sem).wait()

        def section(sec, _):
            pltpu.sync_copy(xidx_hbm.at[c, s, sec], xidx)
            pltpu.sync_copy(eaidx_hbm.at[c, s, sec], eaidx)
            pltpu.sync_copy(dst_hbm.at[s, sec], dsti)
            pltpu.async_copy(x2_hbm.at[xidx.at[0]], xrows.at[0], sem_x)
            pltpu.async_copy(ea2_hbm.at[eaidx.at[0]], earows.at[0], sem_e)

            def half_step(b, p, sem_s):
                @pl.when(b + 1 < BPS)
                def _():
                    q = 1 - p
                    pltpu.async_copy(
                        x2_hbm.at[xidx.at[b + 1]], xrows.at[q], sem_x)
                    pltpu.async_copy(
                        ea2_hbm.at[eaidx.at[b + 1]], earows.at[q], sem_e)
                pltpu.make_async_copy(
                    x2_hbm.at[xidx.at[b]], xrows.at[p], sem_x).wait()
                pltpu.make_async_copy(
                    ea2_hbm.at[eaidx.at[b]], earows.at[p], sem_e).wait()

                @pl.when(b >= 2)
                def _():
                    drain_scatter(p, sem_s)

                xr, er, mg = xrows.at[p], earows.at[p], msg.at[p]

                def comp(e, _):
                    for u in range(4):
                        for kq in range(HALF // 16):
                            sl = pl.ds(kq * 16, 16)
                            mg[e * 4 + u, sl] = jnp.maximum(
                                xr[e * 4 + u, sl] + er[e * 4 + u, sl], 0.0)
                    return 0
                lax.fori_loop(0, G // 4, comp, 0)

                pltpu.async_copy(mg, acc.at[dsti.at[b]], sem_s, add=True)

            def step(i, _):
                half_step(i * 2, 0, sem_s0)
                half_step(i * 2 + 1, 1, sem_s1)
                return 0
            lax.fori_loop(0, BPS // 2, step, 0)
            drain_scatter(0, sem_s0)
            drain_scatter(1, sem_s1)
            return 0
        lax.fori_loop(0, SEC, section, 0)

        plsc.subcore_barrier()
        pltpu.sync_copy(acc.at[pl.ds(s * RPT, RPT)],
                        out_hbm.at[pl.ds(base, RPT)])

    return k(x2, ea2, xidx5, eaidx5, dst4, zrows)


def _tc_mlp(x, h2, W1, b1, W2, b2):
    BM = 1000

    def body(x_ref, h_ref, w1_ref, b1_ref, w2_ref, b2_ref, o_ref):
        h = jnp.dot(x_ref[:, :HALF] + h_ref[0], w1_ref[:HALF, :],
                    preferred_element_type=jnp.float32)
        h = h + jnp.dot(x_ref[:, HALF:] + h_ref[1], w1_ref[HALF:, :],
                        preferred_element_type=jnp.float32)
        h = jnp.maximum(h + b1_ref[0], 0.0)
        o_ref[...] = jnp.dot(h, w2_ref[...],
                             preferred_element_type=jnp.float32) + b2_ref[0]

    return pl.pallas_call(
        body,
        grid=(N // BM,),
        in_specs=[
            pl.BlockSpec((BM, D), lambda i: (i, 0)),
            pl.BlockSpec((2, BM, HALF), lambda i: (0, i, 0)),
            pl.BlockSpec((D, D), lambda i: (0, 0)),
            pl.BlockSpec((1, D), lambda i: (0, 0)),
            pl.BlockSpec((D, D), lambda i: (0, 0)),
            pl.BlockSpec((1, D), lambda i: (0, 0)),
        ],
        out_specs=pl.BlockSpec((BM, D), lambda i: (i, 0)),
        out_shape=jax.ShapeDtypeStruct((N, D), jnp.float32),
    )(x, h2, W1, b1.reshape(1, D), W2, b2.reshape(1, D))


def kernel(x, edge_index, edge_attr, W1, b1, W2, b2):
    src = edge_index[0].astype(jnp.int32)
    dst = edge_index[1].astype(jnp.int32)
    x2 = x.reshape(N // 8, 8, 2, HALF).transpose(0, 2, 1, 3)
    x2 = x2.reshape(2 * N, HALF)
    ea2 = edge_attr.reshape(E // 8, 8, 2, HALF).transpose(0, 2, 1, 3)
    ea2 = ea2.reshape(2 * E, HALF)
    padn = EP - EC
    srcp = jnp.pad(src.reshape(NS, EC), ((0, 0), (0, padn))).reshape(-1)
    sbase = 2 * (srcp - (srcp % 8)) + (srcp % 8)
    xidx5 = (sbase[None, :] + jnp.array([[0], [8]], jnp.int32)
             ).reshape(NC, NS, SEC, BPS, G)
    e = jnp.arange(E, dtype=jnp.int32)
    ebase = 2 * (e - (e % 8)) + (e % 8)
    ebp = jnp.pad(ebase.reshape(NS, EC), ((0, 0), (0, padn))).reshape(-1)
    eaidx5 = (ebp[None, :] + jnp.array([[0], [8]], jnp.int32)
              ).reshape(NC, NS, SEC, BPS, G)
    dst4 = jnp.pad(dst.reshape(NS, EC), ((0, 0), (0, padn)),
                   constant_values=TRASH).reshape(NS, SEC, BPS, G)
    zrows = jnp.zeros((RPT, HALF), jnp.float32)
    h = _sc_aggregate(x2, ea2, xidx5, eaidx5, dst4, zrows)
    return _tc_mlp(x, h.reshape(NC, NP, HALF), W1, b1, W2, b2)


# speedup vs baseline: 1.2289x; 1.1846x over previous
import functools

import jax
import jax.numpy as jnp
from jax import lax
from jax.experimental import pallas as pl
from jax.experimental.pallas import tpu as pltpu
from jax.experimental.pallas import tpu_sc as plsc

N = 10000
E = 160000
D = 256
HALF = 128
NC = 2
NS = 16
EC = E // NS
G = 80
SEC = 5
BPS = 25
EP = SEC * BPS * G
NP = 10240
RPT = NP // NS
TRASH = NP - 1


def _sc_aggregate(x2, ea2, xidx5, eaidx5, dst4, zrows):
    mesh = plsc.VectorSubcoreMesh(
        core_axis_name="c", subcore_axis_name="s",
        num_cores=NC, num_subcores=NS)

    @functools.partial(
        pl.kernel,
        out_type=jax.ShapeDtypeStruct((NC * NP, HALF), jnp.float32),
        mesh=mesh,
        scratch_types=[
            pltpu.VMEM_SHARED((NP, HALF), jnp.float32),
            pltpu.VMEM((BPS, G), jnp.int32),
            pltpu.VMEM((BPS, G), jnp.int32),
            pltpu.VMEM((BPS, G), jnp.int32),
            pltpu.VMEM((2, G, HALF), jnp.float32),
            pltpu.VMEM((2, G, HALF), jnp.float32),
            pltpu.SemaphoreType.DMA,
            pltpu.SemaphoreType.DMA,
            pltpu.SemaphoreType.DMA,
            pltpu.SemaphoreType.DMA,
        ],
        compiler_params=pltpu.CompilerParams(use_tc_tiling_on_sc=False),
    )
    def k(x2_hbm, ea2_hbm, xidx_hbm, eaidx_hbm, dst_hbm, z_hbm, out_hbm,
          acc, xidx, eaidx, dsti, xrows, earows,
          sem_x, sem_e, sem_s0, sem_s1):
        c = lax.axis_index("c")
        s = lax.axis_index("s")
        base = c * NP + s * RPT

        pltpu.sync_copy(z_hbm, acc.at[pl.ds(s * RPT, RPT)])
        plsc.subcore_barrier()

        def drain_scatter(p, sem):
            pltpu.make_async_copy(
                x2_hbm.at[pl.ds(0, G)], xrows.at[p], sem).wait()

        def section(sec, _):
            pltpu.sync_copy(xidx_hbm.at[c, s, sec], xidx)
            pltpu.sync_copy(eaidx_hbm.at[c, s, sec], eaidx)
            pltpu.sync_copy(dst_hbm.at[s, sec], dsti)
            pltpu.async_copy(x2_hbm.at[xidx.at[0]], xrows.at[0], sem_x)
            pltpu.async_copy(ea2_hbm.at[eaidx.at[0]], earows.at[0], sem_e)

            def half_step(b, p, sem_sp, sem_sq):
                q = 1 - p
                @pl.when(b >= 1)
                def _():
                    drain_scatter(q, sem_sq)
                @pl.when(b + 1 < BPS)
                def _():
                    pltpu.async_copy(
                        x2_hbm.at[xidx.at[b + 1]], xrows.at[q], sem_x)
                    pltpu.async_copy(
                        ea2_hbm.at[eaidx.at[b + 1]], earows.at[q], sem_e)
                pltpu.make_async_copy(
                    x2_hbm.at[xidx.at[b]], xrows.at[p], sem_x).wait()
                pltpu.make_async_copy(
                    ea2_hbm.at[eaidx.at[b]], earows.at[p], sem_e).wait()

                xr, er = xrows.at[p], earows.at[p]

                def comp(e, _):
                    for u in range(4):
                        for kq in range(HALF // 16):
                            sl = pl.ds(kq * 16, 16)
                            xr[e * 4 + u, sl] = jnp.maximum(
                                xr[e * 4 + u, sl] + er[e * 4 + u, sl], 0.0)
                    return 0
                lax.fori_loop(0, G // 4, comp, 0)

                pltpu.async_copy(xr, acc.at[dsti.at[b]], sem_sp, add=True)

            def step(i, _):
                half_step(i * 2, 0, sem_s0, sem_s1)
                half_step(i * 2 + 1, 1, sem_s1, sem_s0)
                return 0
            lax.fori_loop(0, BPS // 2, step, 0)
            half_step(BPS - 1, 0, sem_s0, sem_s1)
            drain_scatter(0, sem_s0)
            return 0
        lax.fori_loop(0, SEC, section, 0)

        plsc.subcore_barrier()
        pltpu.sync_copy(acc.at[pl.ds(s * RPT, RPT)],
                        out_hbm.at[pl.ds(base, RPT)])

    return k(x2, ea2, xidx5, eaidx5, dst4, zrows)


def _tc_mlp(x, h2, W1, b1, W2, b2):
    BM = 1000

    def body(x_ref, h_ref, w1_ref, b1_ref, w2_ref, b2_ref, o_ref):
        h = jnp.dot(x_ref[:, :HALF] + h_ref[0], w1_ref[:HALF, :],
                    preferred_element_type=jnp.float32)
        h = h + jnp.dot(x_ref[:, HALF:] + h_ref[1], w1_ref[HALF:, :],
                        preferred_element_type=jnp.float32)
        h = jnp.maximum(h + b1_ref[0], 0.0)
        o_ref[...] = jnp.dot(h, w2_ref[...],
                             preferred_element_type=jnp.float32) + b2_ref[0]

    return pl.pallas_call(
        body,
        grid=(N // BM,),
        in_specs=[
            pl.BlockSpec((BM, D), lambda i: (i, 0)),
            pl.BlockSpec((2, BM, HALF), lambda i: (0, i, 0)),
            pl.BlockSpec((D, D), lambda i: (0, 0)),
            pl.BlockSpec((1, D), lambda i: (0, 0)),
            pl.BlockSpec((D, D), lambda i: (0, 0)),
            pl.BlockSpec((1, D), lambda i: (0, 0)),
        ],
        out_specs=pl.BlockSpec((BM, D), lambda i: (i, 0)),
        out_shape=jax.ShapeDtypeStruct((N, D), jnp.float32),
    )(x, h2, W1, b1.reshape(1, D), W2, b2.reshape(1, D))


def kernel(x, edge_index, edge_attr, W1, b1, W2, b2):
    src = edge_index[0].astype(jnp.int32)
    dst = edge_index[1].astype(jnp.int32)
    x2 = x.reshape(N // 8, 8, 2, HALF).transpose(0, 2, 1, 3)
    x2 = x2.reshape(2 * N, HALF)
    ea2 = edge_attr.reshape(E // 8, 8, 2, HALF).transpose(0, 2, 1, 3)
    ea2 = ea2.reshape(2 * E, HALF)
    padn = EP - EC
    srcp = jnp.pad(src.reshape(NS, EC), ((0, 0), (0, padn))).reshape(-1)
    sbase = 2 * (srcp - (srcp % 8)) + (srcp % 8)
    xidx5 = (sbase[None, :] + jnp.array([[0], [8]], jnp.int32)
             ).reshape(NC, NS, SEC, BPS, G)
    e = jnp.arange(E, dtype=jnp.int32)
    ebase = 2 * (e - (e % 8)) + (e % 8)
    ebp = jnp.pad(ebase.reshape(NS, EC), ((0, 0), (0, padn))).reshape(-1)
    eaidx5 = (ebp[None, :] + jnp.array([[0], [8]], jnp.int32)
              ).reshape(NC, NS, SEC, BPS, G)
    dst4 = jnp.pad(dst.reshape(NS, EC), ((0, 0), (0, padn)),
                   constant_values=TRASH).reshape(NS, SEC, BPS, G)
    zrows = jnp.zeros((RPT, HALF), jnp.float32)
    h = _sc_aggregate(x2, ea2, xidx5, eaidx5, dst4, zrows)
    return _tc_mlp(x, h.reshape(NC, NP, HALF), W1, b1, W2, b2)


# speedup vs baseline: 1.2503x; 1.0175x over previous
import functools

import jax
import jax.numpy as jnp
from jax import lax
from jax.experimental import pallas as pl
from jax.experimental.pallas import tpu as pltpu
from jax.experimental.pallas import tpu_sc as plsc

N = 10000
E = 160000
D = 256
HALF = 128
NC = 2
NS = 16
EC = E // NS
G = 80
SEC = 5
BPS = 25
EP = SEC * BPS * G
NP = 10240
RPT = NP // NS
TRASH = NP - 1


def _sc_aggregate(x2, ea4, xidx5, dst4, zrows):
    mesh = plsc.VectorSubcoreMesh(
        core_axis_name="c", subcore_axis_name="s",
        num_cores=NC, num_subcores=NS)

    @functools.partial(
        pl.kernel,
        out_type=jax.ShapeDtypeStruct((NC * NP, HALF), jnp.float32),
        mesh=mesh,
        scratch_types=[
            pltpu.VMEM_SHARED((NP, HALF), jnp.float32),
            pltpu.VMEM((BPS, G), jnp.int32),
            pltpu.VMEM((BPS, G), jnp.int32),
            pltpu.VMEM((2, G, HALF), jnp.float32),
            pltpu.VMEM((2, G // 8, 1, 8, HALF), jnp.float32),
            pltpu.SemaphoreType.DMA,
            pltpu.SemaphoreType.DMA,
            pltpu.SemaphoreType.DMA,
            pltpu.SemaphoreType.DMA,
        ],
        compiler_params=pltpu.CompilerParams(use_tc_tiling_on_sc=False),
    )
    def k(x2_hbm, ea4_hbm, xidx_hbm, dst_hbm, z_hbm, out_hbm,
          acc, xidx, dsti, xrows, earows,
          sem_x, sem_e, sem_s0, sem_s1):
        c = lax.axis_index("c")
        s = lax.axis_index("s")
        base = c * NP + s * RPT

        pltpu.sync_copy(z_hbm, acc.at[pl.ds(s * RPT, RPT)])
        plsc.subcore_barrier()

        def drain_scatter(p, sem):
            pltpu.make_async_copy(
                x2_hbm.at[pl.ds(0, G)], xrows.at[p], sem).wait()

        def section(sec, _):
            pltpu.sync_copy(xidx_hbm.at[c, s, sec], xidx)
            pltpu.sync_copy(dst_hbm.at[s, sec], dsti)
            tsec = s * (EC // 8) + sec * (BPS * G // 8)

            def ea_src(b):
                return ea4_hbm.at[pl.ds(tsec + b * (G // 8), G // 8),
                                  pl.ds(c, 1)]

            pltpu.async_copy(x2_hbm.at[xidx.at[0]], xrows.at[0], sem_x)
            pltpu.async_copy(ea_src(0), earows.at[0], sem_e)

            def half_step(b, p, sem_sp, sem_sq):
                q = 1 - p
                @pl.when(b >= 1)
                def _():
                    drain_scatter(q, sem_sq)
                @pl.when(b + 1 < BPS)
                def _():
                    pltpu.async_copy(
                        x2_hbm.at[xidx.at[b + 1]], xrows.at[q], sem_x)
                    pltpu.async_copy(ea_src(b + 1), earows.at[q], sem_e)
                pltpu.make_async_copy(
                    x2_hbm.at[xidx.at[b]], xrows.at[p], sem_x).wait()
                pltpu.make_async_copy(
                    ea_src(b), earows.at[p], sem_e).wait()

                xr, er = xrows.at[p], earows.at[p]

                def comp(blk, _):
                    for u in range(8):
                        for kq in range(HALF // 16):
                            sl = pl.ds(kq * 16, 16)
                            xr[blk * 8 + u, sl] = jnp.maximum(
                                xr[blk * 8 + u, sl] + er[blk, 0, u, sl], 0.0)
                    return 0
                lax.fori_loop(0, G // 8, comp, 0)

                pltpu.async_copy(xr, acc.at[dsti.at[b]], sem_sp, add=True)

            def step(i, _):
                half_step(i * 2, 0, sem_s0, sem_s1)
                half_step(i * 2 + 1, 1, sem_s1, sem_s0)
                return 0
            lax.fori_loop(0, BPS // 2, step, 0)
            half_step(BPS - 1, 0, sem_s0, sem_s1)
            drain_scatter(0, sem_s0)
            return 0
        lax.fori_loop(0, SEC, section, 0)

        plsc.subcore_barrier()
        pltpu.sync_copy(acc.at[pl.ds(s * RPT, RPT)],
                        out_hbm.at[pl.ds(base, RPT)])

    return k(x2, ea4, xidx5, dst4, zrows)


def _tc_mlp(x, h2, W1, b1, W2, b2):
    BM = 1000

    def body(x_ref, h_ref, w1_ref, b1_ref, w2_ref, b2_ref, o_ref):
        h = jnp.dot(x_ref[:, :HALF] + h_ref[0], w1_ref[:HALF, :],
                    preferred_element_type=jnp.float32)
        h = h + jnp.dot(x_ref[:, HALF:] + h_ref[1], w1_ref[HALF:, :],
                        preferred_element_type=jnp.float32)
        h = jnp.maximum(h + b1_ref[0], 0.0)
        o_ref[...] = jnp.dot(h, w2_ref[...],
                             preferred_element_type=jnp.float32) + b2_ref[0]

    return pl.pallas_call(
        body,
        grid=(N // BM,),
        in_specs=[
            pl.BlockSpec((BM, D), lambda i: (i, 0)),
            pl.BlockSpec((2, BM, HALF), lambda i: (0, i, 0)),
            pl.BlockSpec((D, D), lambda i: (0, 0)),
            pl.BlockSpec((1, D), lambda i: (0, 0)),
            pl.BlockSpec((D, D), lambda i: (0, 0)),
            pl.BlockSpec((1, D), lambda i: (0, 0)),
        ],
        out_specs=pl.BlockSpec((BM, D), lambda i: (i, 0)),
        out_shape=jax.ShapeDtypeStruct((N, D), jnp.float32),
    )(x, h2, W1, b1.reshape(1, D), W2, b2.reshape(1, D))


def kernel(x, edge_index, edge_attr, W1, b1, W2, b2):
    src = edge_index[0].astype(jnp.int32)
    dst = edge_index[1].astype(jnp.int32)
    x2 = x.reshape(N // 8, 8, 2, HALF).transpose(0, 2, 1, 3)
    x2 = x2.reshape(2 * N, HALF)
    ea4 = edge_attr.reshape(E // 8, 8, 2, HALF).transpose(0, 2, 1, 3)
    padn = EP - EC
    srcp = jnp.pad(src.reshape(NS, EC), ((0, 0), (0, padn))).reshape(-1)
    sbase = 2 * (srcp - (srcp % 8)) + (srcp % 8)
    xidx5 = (sbase[None, :] + jnp.array([[0], [8]], jnp.int32)
             ).reshape(NC, NS, SEC, BPS, G)
    dst4 = jnp.pad(dst.reshape(NS, EC), ((0, 0), (0, padn)),
                   constant_values=TRASH).reshape(NS, SEC, BPS, G)
    zrows = jnp.zeros((RPT, HALF), jnp.float32)
    h = _sc_aggregate(x2, ea4, xidx5, dst4, zrows)
    return _tc_mlp(x, h.reshape(NC, NP, HALF), W1, b1, W2, b2)
